# baseline (speedup 1.0000x reference)
import jax
import jax.numpy as jnp
import numpy as np
from jax.experimental import pallas as pl
from jax.experimental.pallas import tpu as pltpu

D_INP = 36
D_MODEL = 144
D_OB = 4
MAX_LEN = 215
D_PE = 16
D_STATIC = 9
BATCH = 128
N_STEP = 215
F_DIM = N_STEP * D_OB

BS = 16
ROWS = BS * D_INP
_INV_SQRT_D = np.float32(1.0 / np.sqrt(F_DIM))


def _fused_kernel(obs_ref, times_ref, len_ref, lenr_ref, static_ref,
                  rexp_ref, ru_ref,
                  wq1_ref, wk1_ref, wv1_ref, wq2_ref, wk2_ref, wv2_ref,
                  adjs_ref, bias_ref, embw_ref, embb_ref, sel_ref, invts_ref,
                  tof_ref, tio_ref,
                  pooled_ref, pe_ref, emb_ref, alpha_ref,
                  wq1_s, wk1_s, wv1_s, wq2_s, wk2_s, wv2_s):
    f32 = jnp.float32
    bf16 = jnp.bfloat16

    @pl.when(pl.program_id(0) == 0)
    def _cast_weights():
        wq1_s[...] = wq1_ref[...].astype(bf16)
        wk1_s[...] = wk1_ref[...].astype(bf16)
        wv1_s[...] = wv1_ref[...].astype(bf16)
        wq2_s[...] = wq2_ref[...].astype(bf16)
        wk2_s[...] = wk2_ref[...].astype(bf16)
        wv2_s[...] = wv2_ref[...].astype(bf16)

    aexp = jax.lax.dot_general(
        obs_ref[...], rexp_ref[...], (((1,), (0,)), ((), ())),
        preferred_element_type=f32)
    x = jnp.maximum(aexp * ru_ref[...], 0.0).astype(bf16)

    def proj(xin, w_ref):
        return jax.lax.dot_general(
            xin, w_ref[...], (((1,), (0,)), ((), ())),
            preferred_element_type=f32).astype(bf16)

    def attn(xin, wq_ref, wk_ref, wv_ref, edge_w, cast_out=True):
        q = proj(xin, wq_ref)
        k = proj(xin, wk_ref)
        v = proj(xin, wv_ref)
        s = jax.lax.dot_general(
            q, k, (((1,), (1,)), ((), ())), preferred_element_type=f32)
        s = s * edge_w + bias_ref[...]
        a = jnp.exp(s - jnp.max(s, axis=1, keepdims=True))
        a = a / (jnp.sum(a, axis=1, keepdims=True) + 1e-16)
        out = jax.lax.dot_general(
            a.astype(bf16), v, (((1,), (0,)), ((), ())),
            preferred_element_type=f32)
        if cast_out:
            out = out.astype(bf16)
        return out, a

    x2, a1 = attn(x, wq1_s, wk1_s, wv1_s, adjs_ref[...])
    x3, a2 = attn(x2, wq2_s, wk2_s, wv2_s, a1 * _INV_SQRT_D,
                  cast_out=False)

    for bl in range(BS):
        alpha_ref[bl] = a2[bl * D_INP:(bl + 1) * D_INP,
                           bl * D_INP:(bl + 1) * D_INP]

    lenr = lenr_ref[...]
    keep_f = (tof_ref[...] < lenr).astype(f32)
    masked = x3 * keep_f
    pooled = jax.lax.dot_general(
        masked, sel_ref[...], (((1,), (0,)), ((), ())),
        preferred_element_type=f32)
    pooled_ref[...] = pooled[:, :D_OB] / jnp.maximum(lenr, 1.0)

    lenf = len_ref[...]
    div = jnp.maximum(lenf, 1.0)
    keep_t = ((tio_ref[...] < lenf).astype(f32) / div)
    scaled = times_ref[...][:, None, :] * invts_ref[...][None, :, :]
    sin_s = jnp.sum(jnp.sin(scaled) * keep_t[:, None, :], axis=2)
    cos_s = jnp.sum(jnp.cos(scaled) * keep_t[:, None, :], axis=2)
    pe_ref[...] = jnp.concatenate([sin_s, cos_s], axis=1)

    emb_ref[...] = jax.lax.dot_general(
        static_ref[...], embw_ref[...], (((1,), (0,)), ((), ())),
        preferred_element_type=f32) + embb_ref[...]


def kernel(src, static, times, lengths, R_u, emb_W, emb_b,
           Wq1, Wk1, Wv1, Wq2, Wk2, Wv2, global_structure):
    f32 = jnp.float32
    bf16 = jnp.bfloat16
    obs2d = (src[:, :, :D_INP].transpose(1, 2, 0)
             .reshape(BATCH * D_INP, N_STEP).astype(bf16))
    ru2d = jnp.tile(R_u.reshape(D_INP, D_OB), (BS, N_STEP))
    times_t = times.T
    len_f = lengths.astype(f32).reshape(BATCH, 1)
    len_rows = jnp.repeat(len_f, D_INP, axis=0)
    eye = jnp.eye(D_INP, dtype=f32)
    adjw = global_structure * (1.0 - eye) + eye
    adjs = jnp.tile(adjw.T * _INV_SQRT_D, (BS, BS))
    blk = np.arange(ROWS) // D_INP
    bias = jnp.asarray(
        np.where(blk[:, None] == blk[None, :], 0.0, -1e30).astype(np.float32))
    timescales = MAX_LEN ** np.linspace(0, 1, D_PE // 2)
    inv_ts = jnp.asarray(1.0 / timescales, dtype=f32).reshape(D_PE // 2, 1)
    emb_b2 = emb_b.reshape(1, D_INP)
    rexp = jnp.asarray(
        np.equal(np.arange(F_DIM)[None, :] // D_OB,
                 np.arange(N_STEP)[:, None]).astype(np.float32),
        dtype=bf16)
    sel = jnp.asarray(
        np.equal(np.arange(F_DIM)[:, None] % D_OB,
                 np.arange(128)[None, :]).astype(np.float32))
    tof = jnp.asarray((np.arange(F_DIM) // D_OB).astype(np.float32)
                      ).reshape(1, F_DIM)
    tio = jnp.asarray(np.arange(N_STEP, dtype=np.float32)
                      ).reshape(1, N_STEP)

    grid = (BATCH // BS,)
    const = lambda *shape: pl.BlockSpec(shape, lambda i: (0,) * len(shape))
    batched = lambda *shape: pl.BlockSpec(
        shape, lambda i, _n=len(shape): (i,) + (0,) * (_n - 1))

    pooled, pe, emb, alpha = pl.pallas_call(
        _fused_kernel,
        grid=grid,
        in_specs=[
            batched(ROWS, N_STEP),
            batched(BS, N_STEP),
            batched(BS, 1),
            batched(ROWS, 1),
            batched(BS, D_STATIC),
            const(N_STEP, F_DIM),
            const(ROWS, F_DIM),
            const(F_DIM, F_DIM),
            const(F_DIM, F_DIM),
            const(F_DIM, F_DIM),
            const(F_DIM, F_DIM),
            const(F_DIM, F_DIM),
            const(F_DIM, F_DIM),
            const(ROWS, ROWS),
            const(ROWS, ROWS),
            const(D_STATIC, D_INP),
            const(1, D_INP),
            const(F_DIM, 128),
            const(D_PE // 2, 1),
            const(1, F_DIM),
            const(1, N_STEP),
        ],
        out_specs=[
            batched(ROWS, D_OB),
            batched(BS, D_PE),
            batched(BS, D_INP),
            batched(BS, D_INP, D_INP),
        ],
        out_shape=[
            jax.ShapeDtypeStruct((BATCH * D_INP, D_OB), f32),
            jax.ShapeDtypeStruct((BATCH, D_PE), f32),
            jax.ShapeDtypeStruct((BATCH, D_INP), f32),
            jax.ShapeDtypeStruct((BATCH, D_INP, D_INP), f32),
        ],
        scratch_shapes=[pltpu.VMEM((F_DIM, F_DIM), bf16)] * 6,
    )(obs2d, times_t, len_f, len_rows, static, rexp, ru2d,
      Wq1, Wk1, Wv1, Wq2, Wk2, Wv2,
      adjs, bias, emb_W, emb_b2, sel, inv_ts, tof, tio)

    final = jnp.concatenate(
        [pooled.reshape(BATCH, D_MODEL), pe, emb], axis=-1)
    alpha_all = alpha.transpose(0, 2, 1).reshape(BATCH, D_INP * D_INP).T
    return final, alpha_all


# speedup vs baseline: 1.1069x; 1.1069x over previous
import jax
import jax.numpy as jnp
import numpy as np
from jax.experimental import pallas as pl
from jax.experimental.pallas import tpu as pltpu

D_INP = 36
D_MODEL = 144
D_OB = 4
MAX_LEN = 215
D_PE = 16
D_STATIC = 9
BATCH = 128
N_STEP = 215
F_DIM = N_STEP * D_OB

BS = 8
ROWS = BS * D_INP
_INV_SQRT_D = np.float32(1.0 / np.sqrt(F_DIM))


def _fused_kernel(obs_ref, times_ref, len_ref, lenr_ref, static_ref,
                  rexp_ref, ru_ref,
                  wq1_ref, wk1_ref, wv1_ref, wq2_ref, wk2_ref, wv2_ref,
                  adjs_ref, bias_ref, embw_ref, embb_ref, sel_ref, invts_ref,
                  tof_ref, tio_ref,
                  pooled_ref, pe_ref, emb_ref, alpha_ref,
                  wq1_s, wk1_s, wv1_s, wq2_s, wk2_s, wv2_s, aexp_s):
    f32 = jnp.float32
    bf16 = jnp.bfloat16
    i = pl.program_id(0)

    @pl.when(i == 0)
    def _prologue():
        wq1_s[...] = wq1_ref[...].astype(bf16)
        wk1_s[...] = wk1_ref[...].astype(bf16)
        wv1_s[...] = wv1_ref[...].astype(bf16)
        wq2_s[...] = wq2_ref[...].astype(bf16)
        wk2_s[...] = wk2_ref[...].astype(bf16)
        wv2_s[...] = wv2_ref[...].astype(bf16)
        aexp_s[...] = jax.lax.dot_general(
            obs_ref[...], rexp_ref[...], (((0,), (0,)), ((), ())),
            preferred_element_type=f32).astype(bf16)

    aexp = aexp_s[pl.ds(i * ROWS, ROWS), :]
    x = jnp.maximum(aexp * ru_ref[...], 0.0).astype(bf16)

    def proj(xin, w_ref, scale=None):
        out = jax.lax.dot_general(
            xin, w_ref[...], (((1,), (0,)), ((), ())),
            preferred_element_type=f32)
        if scale is not None:
            out = out * scale
        return out.astype(bf16)

    def attn(xin, wq_ref, wk_ref, wv_ref, edge_w, q_scale=None,
             cast_out=True):
        q = proj(xin, wq_ref, q_scale)
        k = proj(xin, wk_ref)
        v = proj(xin, wv_ref)
        s = jax.lax.dot_general(
            q, k, (((1,), (1,)), ((), ())), preferred_element_type=f32)
        s = s * edge_w + bias_ref[...]
        a = jnp.exp(s - jnp.max(s, axis=1, keepdims=True))
        a = a / (jnp.sum(a, axis=1, keepdims=True) + 1e-16)
        out = jax.lax.dot_general(
            a.astype(bf16), v, (((1,), (0,)), ((), ())),
            preferred_element_type=f32)
        if cast_out:
            out = out.astype(bf16)
        return out, a

    x2, a1 = attn(x, wq1_s, wk1_s, wv1_s, adjs_ref[...])
    x3, a2 = attn(x2, wq2_s, wk2_s, wv2_s, a1, q_scale=_INV_SQRT_D,
                  cast_out=False)

    for bl in range(BS):
        alpha_ref[bl] = a2[bl * D_INP:(bl + 1) * D_INP,
                           bl * D_INP:(bl + 1) * D_INP]

    lenr = lenr_ref[...]
    keep_f = (tof_ref[...] < lenr).astype(f32)
    masked = x3 * keep_f
    pooled = jax.lax.dot_general(
        masked, sel_ref[...], (((1,), (0,)), ((), ())),
        preferred_element_type=f32)
    pooled_ref[...] = pooled[:, :D_OB] / jnp.maximum(lenr, 1.0)

    lenf = len_ref[...]
    div = jnp.maximum(lenf, 1.0)
    keep_t = ((tio_ref[...] < lenf).astype(f32) / div)
    scaled = times_ref[...][:, None, :] * invts_ref[...][None, :, :]
    sin_s = jnp.sum(jnp.sin(scaled) * keep_t[:, None, :], axis=2)
    cos_s = jnp.sum(jnp.cos(scaled) * keep_t[:, None, :], axis=2)
    pe_ref[...] = jnp.concatenate([sin_s, cos_s], axis=1)

    emb_ref[...] = jax.lax.dot_general(
        static_ref[...], embw_ref[...], (((1,), (0,)), ((), ())),
        preferred_element_type=f32) + embb_ref[...]


def kernel(src, static, times, lengths, R_u, emb_W, emb_b,
           Wq1, Wk1, Wv1, Wq2, Wk2, Wv2, global_structure):
    f32 = jnp.float32
    bf16 = jnp.bfloat16
    obs_tb = src[:, :, :D_INP].reshape(N_STEP, BATCH * D_INP).astype(bf16)
    ru2d = jnp.tile(R_u.reshape(D_INP, D_OB), (BS, N_STEP))
    times_t = times.T
    len_f = lengths.astype(f32).reshape(BATCH, 1)
    len_rows = jnp.repeat(len_f, D_INP, axis=0)
    eye = jnp.eye(D_INP, dtype=f32)
    adjw = global_structure * (1.0 - eye) + eye
    adjs = jnp.tile(adjw.T * _INV_SQRT_D, (BS, BS))
    blk = np.arange(ROWS) // D_INP
    bias = jnp.asarray(
        np.where(blk[:, None] == blk[None, :], 0.0, -1e30).astype(np.float32))
    timescales = MAX_LEN ** np.linspace(0, 1, D_PE // 2)
    inv_ts = jnp.asarray(1.0 / timescales, dtype=f32).reshape(D_PE // 2, 1)
    emb_b2 = emb_b.reshape(1, D_INP)
    rexp = jnp.asarray(
        np.equal(np.arange(F_DIM)[None, :] // D_OB,
                 np.arange(N_STEP)[:, None]).astype(np.float32),
        dtype=bf16)
    sel = jnp.asarray(
        np.equal(np.arange(F_DIM)[:, None] % D_OB,
                 np.arange(128)[None, :]).astype(np.float32))
    tof = jnp.asarray((np.arange(F_DIM) // D_OB).astype(np.float32)
                      ).reshape(1, F_DIM)
    tio = jnp.asarray(np.arange(N_STEP, dtype=np.float32)
                      ).reshape(1, N_STEP)

    grid = (BATCH // BS,)
    const = lambda *shape: pl.BlockSpec(shape, lambda i: (0,) * len(shape))
    batched = lambda *shape: pl.BlockSpec(
        shape, lambda i, _n=len(shape): (i,) + (0,) * (_n - 1))

    pooled, pe, emb, alpha = pl.pallas_call(
        _fused_kernel,
        grid=grid,
        in_specs=[
            const(N_STEP, BATCH * D_INP),
            batched(BS, N_STEP),
            batched(BS, 1),
            batched(ROWS, 1),
            batched(BS, D_STATIC),
            const(N_STEP, F_DIM),
            const(ROWS, F_DIM),
            const(F_DIM, F_DIM),
            const(F_DIM, F_DIM),
            const(F_DIM, F_DIM),
            const(F_DIM, F_DIM),
            const(F_DIM, F_DIM),
            const(F_DIM, F_DIM),
            const(ROWS, ROWS),
            const(ROWS, ROWS),
            const(D_STATIC, D_INP),
            const(1, D_INP),
            const(F_DIM, 128),
            const(D_PE // 2, 1),
            const(1, F_DIM),
            const(1, N_STEP),
        ],
        out_specs=[
            batched(ROWS, D_OB),
            batched(BS, D_PE),
            batched(BS, D_INP),
            batched(BS, D_INP, D_INP),
        ],
        out_shape=[
            jax.ShapeDtypeStruct((BATCH * D_INP, D_OB), f32),
            jax.ShapeDtypeStruct((BATCH, D_PE), f32),
            jax.ShapeDtypeStruct((BATCH, D_INP), f32),
            jax.ShapeDtypeStruct((BATCH, D_INP, D_INP), f32),
        ],
        scratch_shapes=([pltpu.VMEM((F_DIM, F_DIM), bf16)] * 6
                        + [pltpu.VMEM((BATCH * D_INP, F_DIM), bf16)]),
    )(obs_tb, times_t, len_f, len_rows, static, rexp, ru2d,
      Wq1, Wk1, Wv1, Wq2, Wk2, Wv2,
      adjs, bias, emb_W, emb_b2, sel, inv_ts, tof, tio)

    final = jnp.concatenate(
        [pooled.reshape(BATCH, D_MODEL), pe, emb], axis=-1)
    alpha_all = alpha.transpose(0, 2, 1).reshape(BATCH, D_INP * D_INP).T
    return final, alpha_all


# speedup vs baseline: 1.1313x; 1.0220x over previous
import jax
import jax.numpy as jnp
import numpy as np
from jax.experimental import pallas as pl
from jax.experimental.pallas import tpu as pltpu

D_INP = 36
D_MODEL = 144
D_OB = 4
MAX_LEN = 215
D_PE = 16
D_STATIC = 9
BATCH = 128
N_STEP = 215
F_DIM = N_STEP * D_OB

BS = 8
ROWS = BS * D_INP
_INV_SQRT_D = np.float32(1.0 / np.sqrt(F_DIM))


def _fused_kernel(obs_ref, times_ref, len_ref, lenr_ref, static_ref,
                  rexp_ref, ru_ref,
                  wq1_ref, wk1_ref, wv1_ref, wq2_ref, wk2_ref, wv2_ref,
                  adjs_ref, bias_ref, embw_ref, embb_ref, sel_ref, invts_ref,
                  tof_ref, tio_ref,
                  pooled_ref, pe_ref, emb_ref, alpha_ref,
                  wq1_s, wk1_s, wv1_s, wq2_s, wk2_s, wv2_s):
    f32 = jnp.float32
    bf16 = jnp.bfloat16

    @pl.when(pl.program_id(0) == 0)
    def _cast_weights():
        wq1_s[...] = wq1_ref[...].astype(bf16)
        wk1_s[...] = wk1_ref[...].astype(bf16)
        wv1_s[...] = wv1_ref[...].astype(bf16)
        wq2_s[...] = wq2_ref[...].astype(bf16)
        wk2_s[...] = wk2_ref[...].astype(bf16)
        wv2_s[...] = wv2_ref[...].astype(bf16)

    aexp = jax.lax.dot_general(
        obs_ref[...], rexp_ref[...], (((1,), (0,)), ((), ())),
        preferred_element_type=f32)
    x = jnp.maximum(aexp * ru_ref[...], 0.0).astype(bf16)

    def proj(xin, w_ref, scale=None):
        out = jax.lax.dot_general(
            xin, w_ref[...], (((1,), (0,)), ((), ())),
            preferred_element_type=f32)
        if scale is not None:
            out = out * scale
        return out.astype(bf16)

    def attn(xin, wq_ref, wk_ref, wv_ref, edge_w, q_scale=None,
             cast_out=True):
        q = proj(xin, wq_ref, q_scale)
        k = proj(xin, wk_ref)
        v = proj(xin, wv_ref)
        s = jax.lax.dot_general(
            q, k, (((1,), (1,)), ((), ())), preferred_element_type=f32)
        s = s * edge_w + bias_ref[...]
        a = jnp.exp(s - jnp.max(s, axis=1, keepdims=True))
        a = a / (jnp.sum(a, axis=1, keepdims=True) + 1e-16)
        out = jax.lax.dot_general(
            a.astype(bf16), v, (((1,), (0,)), ((), ())),
            preferred_element_type=f32)
        if cast_out:
            out = out.astype(bf16)
        return out, a

    x2, a1 = attn(x, wq1_s, wk1_s, wv1_s, adjs_ref[...])
    x3, a2 = attn(x2, wq2_s, wk2_s, wv2_s, a1, q_scale=_INV_SQRT_D,
                  cast_out=False)

    for bl in range(BS):
        alpha_ref[bl] = a2[bl * D_INP:(bl + 1) * D_INP,
                           bl * D_INP:(bl + 1) * D_INP]

    lenr = lenr_ref[...]
    keep_f = (tof_ref[...] < lenr).astype(f32)
    masked = x3 * keep_f
    pooled = jax.lax.dot_general(
        masked, sel_ref[...], (((1,), (0,)), ((), ())),
        preferred_element_type=f32)
    pooled_ref[...] = pooled[:, :D_OB] / jnp.maximum(lenr, 1.0)

    lenf = len_ref[...]
    div = jnp.maximum(lenf, 1.0)
    keep_t = ((tio_ref[...] < lenf).astype(f32) / div)
    scaled = times_ref[...][:, None, :] * invts_ref[...][None, :, :]
    sin_s = jnp.sum(jnp.sin(scaled) * keep_t[:, None, :], axis=2)
    cos_s = jnp.sum(jnp.cos(scaled) * keep_t[:, None, :], axis=2)
    pe_ref[...] = jnp.concatenate([sin_s, cos_s], axis=1)

    emb_ref[...] = jax.lax.dot_general(
        static_ref[...], embw_ref[...], (((1,), (0,)), ((), ())),
        preferred_element_type=f32) + embb_ref[...]


def kernel(src, static, times, lengths, R_u, emb_W, emb_b,
           Wq1, Wk1, Wv1, Wq2, Wk2, Wv2, global_structure):
    f32 = jnp.float32
    bf16 = jnp.bfloat16
    obs2d = (src[:, :, :D_INP].transpose(1, 2, 0)
             .reshape(BATCH * D_INP, N_STEP).astype(bf16))
    ru2d = jnp.tile(R_u.reshape(D_INP, D_OB), (BS, N_STEP))
    times_t = times.T
    len_f = lengths.astype(f32).reshape(BATCH, 1)
    len_rows = jnp.repeat(len_f, D_INP, axis=0)
    eye = jnp.eye(D_INP, dtype=f32)
    adjw = global_structure * (1.0 - eye) + eye
    adjs = jnp.tile(adjw.T * _INV_SQRT_D, (BS, BS))
    blk = np.arange(ROWS) // D_INP
    bias = jnp.asarray(
        np.where(blk[:, None] == blk[None, :], 0.0, -1e30).astype(np.float32))
    timescales = MAX_LEN ** np.linspace(0, 1, D_PE // 2)
    inv_ts = jnp.asarray(1.0 / timescales, dtype=f32).reshape(D_PE // 2, 1)
    emb_b2 = emb_b.reshape(1, D_INP)
    rexp = jnp.asarray(
        np.equal(np.arange(F_DIM)[None, :] // D_OB,
                 np.arange(N_STEP)[:, None]).astype(np.float32),
        dtype=bf16)
    sel = jnp.asarray(
        np.equal(np.arange(F_DIM)[:, None] % D_OB,
                 np.arange(128)[None, :]).astype(np.float32))
    tof = jnp.asarray((np.arange(F_DIM) // D_OB).astype(np.float32)
                      ).reshape(1, F_DIM)
    tio = jnp.asarray(np.arange(N_STEP, dtype=np.float32)
                      ).reshape(1, N_STEP)

    grid = (BATCH // BS,)
    const = lambda *shape: pl.BlockSpec(shape, lambda i: (0,) * len(shape))
    batched = lambda *shape: pl.BlockSpec(
        shape, lambda i, _n=len(shape): (i,) + (0,) * (_n - 1))

    pooled, pe, emb, alpha = pl.pallas_call(
        _fused_kernel,
        grid=grid,
        in_specs=[
            batched(ROWS, N_STEP),
            batched(BS, N_STEP),
            batched(BS, 1),
            batched(ROWS, 1),
            batched(BS, D_STATIC),
            const(N_STEP, F_DIM),
            const(ROWS, F_DIM),
            const(F_DIM, F_DIM),
            const(F_DIM, F_DIM),
            const(F_DIM, F_DIM),
            const(F_DIM, F_DIM),
            const(F_DIM, F_DIM),
            const(F_DIM, F_DIM),
            const(ROWS, ROWS),
            const(ROWS, ROWS),
            const(D_STATIC, D_INP),
            const(1, D_INP),
            const(F_DIM, 128),
            const(D_PE // 2, 1),
            const(1, F_DIM),
            const(1, N_STEP),
        ],
        out_specs=[
            batched(ROWS, D_OB),
            batched(BS, D_PE),
            batched(BS, D_INP),
            batched(BS, D_INP, D_INP),
        ],
        out_shape=[
            jax.ShapeDtypeStruct((BATCH * D_INP, D_OB), f32),
            jax.ShapeDtypeStruct((BATCH, D_PE), f32),
            jax.ShapeDtypeStruct((BATCH, D_INP), f32),
            jax.ShapeDtypeStruct((BATCH, D_INP, D_INP), f32),
        ],
        scratch_shapes=[pltpu.VMEM((F_DIM, F_DIM), bf16)] * 6,
    )(obs2d, times_t, len_f, len_rows, static, rexp, ru2d,
      Wq1, Wk1, Wv1, Wq2, Wk2, Wv2,
      adjs, bias, emb_W, emb_b2, sel, inv_ts, tof, tio)

    final = jnp.concatenate(
        [pooled.reshape(BATCH, D_MODEL), pe, emb], axis=-1)
    alpha_all = alpha.transpose(0, 2, 1).reshape(BATCH, D_INP * D_INP).T
    return final, alpha_all
